# baseline (speedup 1.0000x reference)
import functools

import jax
import jax.numpy as jnp
from jax import lax
from jax.experimental import pallas as pl
from jax.experimental.pallas import tpu as pltpu
from jax.experimental.pallas import tpu_sc as plsc

L = 512
D_PAIR = 192
NTAB = 66 * 10 * 2
P = L * L

_NC = 2
_NS = 16
NW = _NC * _NS
PER_W = P // NW
ROWS_PER_W = PER_W * 2
CHUNK = 128
BLOCK = 256
NBUF = 2
NGROUP = ROWS_PER_W // (BLOCK * NBUF)


def _tc_body(seq_r, seq_c, idx_r, idx_c, bf, sc, wr, wa, wc, ca_out, cb_out,
             ta_out, tb_out):
    f32 = jnp.float32
    sm_r = seq_r[...] >= 32
    sm_c = seq_c[...] >= 32
    sm2 = jnp.logical_and(sm_c, sm_r)

    dres = jnp.clip(idx_r[...] - idx_c[...], -32, 32) + 32
    ib_res = jnp.where(sm2, 65, dres)

    bfm = bf[0]
    adj = jnp.logical_and(bfm > 0, bfm < 5).astype(f32)
    ir = lax.broadcasted_iota(jnp.int32, (L, L), 0)
    ic = lax.broadcasted_iota(jnp.int32, (L, L), 1)
    eye = ir == ic
    dist = jnp.where(adj > 0, 1.0, 9.0)
    dist = jnp.where(eye, 0.0, dist)
    reach = jnp.minimum(adj + eye.astype(f32), 1.0)
    cur = reach
    for k in range(2, 9):
        cur = (jnp.dot(cur, reach, preferred_element_type=f32) > 0).astype(f32)
        dist = jnp.where((cur > 0) & (dist >= 9.0), float(k), dist)
    atom_sm = jnp.minimum(dist, 8.0).astype(jnp.int32)
    ib_atom = jnp.where(sm2, atom_sm, 9)

    cidx = ib_res * 20 + ib_atom * 2 + sc[0]
    ca_out[...] = cidx * 2
    cb_out[...] = cidx * 2 + 1

    rid = lax.broadcasted_iota(jnp.int32, (NTAB, 1), 0)
    a = rid // 20
    b = (rid % 20) // 2
    c = rid % 2
    oh_a = (lax.broadcasted_iota(jnp.int32, (NTAB, 66), 1) == a).astype(f32)
    oh_b = (lax.broadcasted_iota(jnp.int32, (NTAB, 10), 1) == b).astype(f32)
    oh_c = (lax.broadcasted_iota(jnp.int32, (NTAB, 2), 1) == c).astype(f32)
    hi = lax.Precision.HIGHEST
    table = (
        jnp.dot(oh_a, wr[...], precision=hi, preferred_element_type=f32)
        + jnp.dot(oh_b, wa[...], precision=hi, preferred_element_type=f32)
        + jnp.dot(oh_c, wc[...], precision=hi, preferred_element_type=f32))
    ta_out[...] = table[:, :128]
    tb_out[...] = jnp.concatenate(
        [table[:, 128:], jnp.zeros((NTAB, 64), f32)], axis=1)


def _index_and_table(seq_r, seq_c, idx_r, idx_c, bond_feats, same_chain,
                     emb_res_w, emb_atom_w, emb_chain_w):
    return pl.pallas_call(
        _tc_body,
        out_shape=(
            jax.ShapeDtypeStruct((L, L), jnp.int32),
            jax.ShapeDtypeStruct((L, L), jnp.int32),
            jax.ShapeDtypeStruct((NTAB, 128), jnp.float32),
            jax.ShapeDtypeStruct((NTAB, 128), jnp.float32),
        ),
    )(seq_r, seq_c, idx_r, idx_c, bond_feats, same_chain, emb_res_w,
      emb_atom_w, emb_chain_w)


def _sc_gather_body(idx2_hbm, table_hbm, out_hbm, idx_v, rows_v, gsem, wsem):
    wid = lax.axis_index("s") * _NC + lax.axis_index("c")
    base = wid * ROWS_PER_W
    pltpu.sync_copy(idx2_hbm.at[pl.ds(base, ROWS_PER_W)], idx_v)

    def body(t, carry):
        for b in range(NBUF):
            off = pl.multiple_of((t * NBUF + b) * BLOCK, BLOCK)

            @pl.when(t > 0)
            def _(b=b):
                pltpu.make_async_copy(
                    rows_v.at[b], out_hbm.at[pl.ds(0, BLOCK)],
                    wsem.at[b]).wait()

            handles = []
            for k in range(BLOCK // CHUNK):
                handles.append(pltpu.async_copy(
                    table_hbm.at[idx_v.at[pl.ds(off + k * CHUNK, CHUNK)]],
                    rows_v.at[b, pl.ds(k * CHUNK, CHUNK)],
                    gsem.at[b]))
            for h in handles:
                h.wait()
            pltpu.async_copy(rows_v.at[b],
                             out_hbm.at[pl.ds(base + off, BLOCK)],
                             wsem.at[b])
        return carry

    lax.fori_loop(0, NGROUP, body, 0)
    for b in range(NBUF):
        pltpu.make_async_copy(rows_v.at[b], out_hbm.at[pl.ds(0, BLOCK)],
                              wsem.at[b]).wait()


_SC_GATHER_CACHE = []


def _sc_gather(idx2, table2):
    if not _SC_GATHER_CACHE:
        _SC_GATHER_CACHE.append(functools.partial(
            pl.kernel,
            mesh=plsc.VectorSubcoreMesh(core_axis_name="c",
                                        subcore_axis_name="s"),
            out_type=jax.ShapeDtypeStruct((2 * P, 128), jnp.float32),
            scratch_types=[
                pltpu.VMEM((ROWS_PER_W,), jnp.int32),
                pltpu.VMEM((NBUF, BLOCK, 128), jnp.float32),
                pltpu.SemaphoreType.DMA((NBUF,)),
                pltpu.SemaphoreType.DMA((NBUF,)),
            ],
            compiler_params=pltpu.CompilerParams(use_tc_tiling_on_sc=True),
        )(_sc_gather_body))
    return _SC_GATHER_CACHE[0](idx2, table2)


def kernel(seq, idx, bond_feats, same_chain, emb_res_w, emb_atom_w,
           emb_chain_w):
    seq = seq.astype(jnp.int32)
    idx = idx.astype(jnp.int32)
    bond_feats = bond_feats.astype(jnp.int32)
    same_chain = same_chain.astype(jnp.int32)
    seq_r = seq.reshape(1, L)
    seq_c = seq.reshape(L, 1)
    idx_r = idx.reshape(1, L)
    idx_c = idx.reshape(L, 1)
    ca, cb, ta, tb = _index_and_table(seq_r, seq_c, idx_r, idx_c, bond_feats,
                                      same_chain, emb_res_w, emb_atom_w,
                                      emb_chain_w)
    table2 = jnp.stack([ta, tb], axis=1).reshape(2 * NTAB, 128)
    a8 = ca.reshape(P // 8, 8)
    b8 = cb.reshape(P // 8, 8)
    idx2 = jnp.concatenate([a8, b8], axis=1).reshape(2 * P)
    out = _sc_gather(idx2, table2)
    x = out.reshape(P // 8, 2, 8, 128)
    first = x[:, 0].reshape(P, 128)
    second = x[:, 1, :, :64].reshape(P, 64)
    return jnp.concatenate([first, second], axis=-1).reshape(1, L, L, D_PAIR)


# speedup vs baseline: 4.3357x; 4.3357x over previous
import functools

import jax
import jax.numpy as jnp
from jax import lax
from jax.experimental import pallas as pl
from jax.experimental.pallas import tpu as pltpu
from jax.experimental.pallas import tpu_sc as plsc

L = 512
D_PAIR = 192
NTAB = 66 * 10 * 2
P = L * L

_NC = 2
_NS = 16
NW = _NC * _NS
PER_W = P // NW
ROWS_PER_W = PER_W * 2
CHUNK = 128
BLOCK = 256
NBUF = 2
NGROUP = ROWS_PER_W // (BLOCK * NBUF)


def _tc_body(seq_r, seq_c, idx_r, idx_c, bf, sc, wr, wa, wc, ca_out, cb_out,
             ta_out, tb_out):
    f32 = jnp.float32
    sm_r = seq_r[...] >= 32
    sm_c = seq_c[...] >= 32
    sm2 = jnp.logical_and(sm_c, sm_r)

    dres = jnp.clip(idx_r[...] - idx_c[...], -32, 32) + 32
    ib_res = jnp.where(sm2, 65, dres)

    bfm = bf[0]
    adj = jnp.logical_and(bfm > 0, bfm < 5).astype(f32)
    ir = lax.broadcasted_iota(jnp.int32, (L, L), 0)
    ic = lax.broadcasted_iota(jnp.int32, (L, L), 1)
    eye = ir == ic
    dist = jnp.where(adj > 0, 1.0, 9.0)
    dist = jnp.where(eye, 0.0, dist)
    reach = jnp.minimum(adj + eye.astype(f32), 1.0)
    cur = reach
    for k in range(2, 9):
        cur = (jnp.dot(cur, reach, preferred_element_type=f32) > 0).astype(f32)
        dist = jnp.where((cur > 0) & (dist >= 9.0), float(k), dist)
    atom_sm = jnp.minimum(dist, 8.0).astype(jnp.int32)
    ib_atom = jnp.where(sm2, atom_sm, 9)

    cidx = ib_res * 20 + ib_atom * 2 + sc[0]
    ca_out[...] = cidx * 2
    cb_out[...] = cidx * 2 + 1

    rid = lax.broadcasted_iota(jnp.int32, (NTAB, 1), 0)
    a = rid // 20
    b = (rid % 20) // 2
    c = rid % 2
    oh_a = (lax.broadcasted_iota(jnp.int32, (NTAB, 66), 1) == a).astype(f32)
    oh_b = (lax.broadcasted_iota(jnp.int32, (NTAB, 10), 1) == b).astype(f32)
    oh_c = (lax.broadcasted_iota(jnp.int32, (NTAB, 2), 1) == c).astype(f32)
    hi = lax.Precision.HIGHEST
    table = (
        jnp.dot(oh_a, wr[...], precision=hi, preferred_element_type=f32)
        + jnp.dot(oh_b, wa[...], precision=hi, preferred_element_type=f32)
        + jnp.dot(oh_c, wc[...], precision=hi, preferred_element_type=f32))
    ta_out[...] = table[:, :128]
    tb_out[...] = jnp.concatenate(
        [table[:, 128:], jnp.zeros((NTAB, 64), f32)], axis=1)


def _index_and_table(seq_r, seq_c, idx_r, idx_c, bond_feats, same_chain,
                     emb_res_w, emb_atom_w, emb_chain_w):
    return pl.pallas_call(
        _tc_body,
        out_shape=(
            jax.ShapeDtypeStruct((L, L), jnp.int32),
            jax.ShapeDtypeStruct((L, L), jnp.int32),
            jax.ShapeDtypeStruct((NTAB, 128), jnp.float32),
            jax.ShapeDtypeStruct((NTAB, 128), jnp.float32),
        ),
    )(seq_r, seq_c, idx_r, idx_c, bond_feats, same_chain, emb_res_w,
      emb_atom_w, emb_chain_w)


def _sc_gather_body(idx2_hbm, table_hbm, out_hbm, idx_v, rows_v, tab_sh,
                    gsem, wsem):
    wid = lax.axis_index("s") * _NC + lax.axis_index("c")
    base = wid * ROWS_PER_W
    pltpu.sync_copy(idx2_hbm.at[pl.ds(base, ROWS_PER_W)], idx_v)

    @pl.when(lax.axis_index("s") == 0)
    def _():
        pltpu.sync_copy(table_hbm, tab_sh)

    plsc.subcore_barrier()

    def body(t, carry):
        for b in range(NBUF):
            off = pl.multiple_of((t * NBUF + b) * BLOCK, BLOCK)

            @pl.when(t > 0)
            def _(b=b):
                pltpu.make_async_copy(
                    rows_v.at[b], out_hbm.at[pl.ds(0, BLOCK)],
                    wsem.at[b]).wait()

            handles = []
            for k in range(BLOCK // CHUNK):
                handles.append(pltpu.async_copy(
                    tab_sh.at[idx_v.at[pl.ds(off + k * CHUNK, CHUNK)]],
                    rows_v.at[b, pl.ds(k * CHUNK, CHUNK)],
                    gsem.at[b]))
            for h in handles:
                h.wait()
            pltpu.async_copy(rows_v.at[b],
                             out_hbm.at[pl.ds(base + off, BLOCK)],
                             wsem.at[b])
        return carry

    lax.fori_loop(0, NGROUP, body, 0)
    for b in range(NBUF):
        pltpu.make_async_copy(rows_v.at[b], out_hbm.at[pl.ds(0, BLOCK)],
                              wsem.at[b]).wait()


_SC_GATHER_CACHE = []


def _sc_gather(idx2, table2):
    if not _SC_GATHER_CACHE:
        _SC_GATHER_CACHE.append(functools.partial(
            pl.kernel,
            mesh=plsc.VectorSubcoreMesh(core_axis_name="c",
                                        subcore_axis_name="s"),
            out_type=jax.ShapeDtypeStruct((2 * P, 128), jnp.float32),
            scratch_types=[
                pltpu.VMEM((ROWS_PER_W,), jnp.int32),
                pltpu.VMEM((NBUF, BLOCK, 128), jnp.float32),
                pltpu.VMEM_SHARED((2 * NTAB, 128), jnp.float32),
                pltpu.SemaphoreType.DMA((NBUF,)),
                pltpu.SemaphoreType.DMA((NBUF,)),
            ],
            compiler_params=pltpu.CompilerParams(use_tc_tiling_on_sc=False),
        )(_sc_gather_body))
    return _SC_GATHER_CACHE[0](idx2, table2)


def kernel(seq, idx, bond_feats, same_chain, emb_res_w, emb_atom_w,
           emb_chain_w):
    seq = seq.astype(jnp.int32)
    idx = idx.astype(jnp.int32)
    bond_feats = bond_feats.astype(jnp.int32)
    same_chain = same_chain.astype(jnp.int32)
    seq_r = seq.reshape(1, L)
    seq_c = seq.reshape(L, 1)
    idx_r = idx.reshape(1, L)
    idx_c = idx.reshape(L, 1)
    ca, cb, ta, tb = _index_and_table(seq_r, seq_c, idx_r, idx_c, bond_feats,
                                      same_chain, emb_res_w, emb_atom_w,
                                      emb_chain_w)
    table2 = jnp.stack([ta, tb], axis=1).reshape(2 * NTAB, 128)
    a8 = ca.reshape(P // 8, 8)
    b8 = cb.reshape(P // 8, 8)
    idx2 = jnp.concatenate([a8, b8], axis=1).reshape(2 * P)
    out = _sc_gather(idx2, table2)
    x = out.reshape(P // 8, 2, 8, 128)
    first = x[:, 0].reshape(P, 128)
    second = x[:, 1, :, :64].reshape(P, 64)
    return jnp.concatenate([first, second], axis=-1).reshape(1, L, L, D_PAIR)


# speedup vs baseline: 5.7572x; 1.3278x over previous
import functools

import jax
import jax.numpy as jnp
from jax import lax
from jax.experimental import pallas as pl
from jax.experimental.pallas import tpu as pltpu
from jax.experimental.pallas import tpu_sc as plsc

L = 512
D_PAIR = 192
NTAB = 66 * 10 * 2
P = L * L

_NC = 2
_NS = 16
NW = _NC * _NS
PER_W = P // NW
ROWS_W = PER_W // L
CHUNK = 128
BLOCK = 128
NBUF = 4
NBLK = PER_W // BLOCK
NGROUP = NBLK // NBUF


def _tc_body(seq_r, seq_c, idx_r, idx_c, bf, sc, wr, wa, wc, cidx_out,
             table_out):
    f32 = jnp.float32
    sm_r = seq_r[...] >= 32
    sm_c = seq_c[...] >= 32
    sm2 = jnp.logical_and(sm_c, sm_r)

    dres = jnp.clip(idx_r[...] - idx_c[...], -32, 32) + 32
    ib_res = jnp.where(sm2, 65, dres)

    bfm = bf[0]
    adj = jnp.logical_and(bfm > 0, bfm < 5).astype(f32)
    ir = lax.broadcasted_iota(jnp.int32, (L, L), 0)
    ic = lax.broadcasted_iota(jnp.int32, (L, L), 1)
    eye = ir == ic
    dist = jnp.where(adj > 0, 1.0, 9.0)
    dist = jnp.where(eye, 0.0, dist)
    reach = jnp.minimum(adj + eye.astype(f32), 1.0)
    cur = reach
    for k in range(2, 9):
        cur = (jnp.dot(cur, reach, preferred_element_type=f32) > 0).astype(f32)
        dist = jnp.where((cur > 0) & (dist >= 9.0), float(k), dist)
    atom_sm = jnp.minimum(dist, 8.0).astype(jnp.int32)
    ib_atom = jnp.where(sm2, atom_sm, 9)

    cidx_out[...] = ib_res * 20 + ib_atom * 2 + sc[0]

    rid = lax.broadcasted_iota(jnp.int32, (NTAB, 1), 0)
    a = rid // 20
    b = (rid % 20) // 2
    c = rid % 2
    oh_a = (lax.broadcasted_iota(jnp.int32, (NTAB, 66), 1) == a).astype(f32)
    oh_b = (lax.broadcasted_iota(jnp.int32, (NTAB, 10), 1) == b).astype(f32)
    oh_c = (lax.broadcasted_iota(jnp.int32, (NTAB, 2), 1) == c).astype(f32)
    hi = lax.Precision.HIGHEST
    table_out[...] = (
        jnp.dot(oh_a, wr[...], precision=hi, preferred_element_type=f32)
        + jnp.dot(oh_b, wa[...], precision=hi, preferred_element_type=f32)
        + jnp.dot(oh_c, wc[...], precision=hi, preferred_element_type=f32))


def _index_and_table(seq_r, seq_c, idx_r, idx_c, bond_feats, same_chain,
                     emb_res_w, emb_atom_w, emb_chain_w):
    return pl.pallas_call(
        _tc_body,
        out_shape=(
            jax.ShapeDtypeStruct((L, L), jnp.int32),
            jax.ShapeDtypeStruct((NTAB, D_PAIR), jnp.float32),
        ),
    )(seq_r, seq_c, idx_r, idx_c, bond_feats, same_chain, emb_res_w,
      emb_atom_w, emb_chain_w)


def _sc_gather_body(cidx_hbm, table_hbm, out_hbm, idx_v, rows_v, tab_sh,
                    gsem, wsem):
    wid = lax.axis_index("s") * _NC + lax.axis_index("c")
    base = wid * PER_W
    i0 = wid * ROWS_W
    pltpu.sync_copy(cidx_hbm.at[pl.ds(base, PER_W)], idx_v)

    @pl.when(lax.axis_index("s") == 0)
    def _():
        pltpu.sync_copy(table_hbm, tab_sh)

    plsc.subcore_barrier()

    def body(t, carry):
        for b in range(NBUF):
            blk = t * NBUF + b
            off = pl.multiple_of(blk * BLOCK, BLOCK)
            i = i0 + blk // (L // BLOCK)
            j0 = (blk % (L // BLOCK)) * BLOCK

            @pl.when(t > 0)
            def _(b=b):
                pltpu.make_async_copy(
                    rows_v.at[b], out_hbm.at[0, i0, pl.ds(0, BLOCK)],
                    wsem.at[b]).wait()

            handles = []
            for k in range(BLOCK // CHUNK):
                handles.append(pltpu.async_copy(
                    tab_sh.at[idx_v.at[pl.ds(off + k * CHUNK, CHUNK)]],
                    rows_v.at[b, pl.ds(k * CHUNK, CHUNK)],
                    gsem.at[b]))
            for h in handles:
                h.wait()
            pltpu.async_copy(rows_v.at[b],
                             out_hbm.at[0, i, pl.ds(j0, BLOCK)],
                             wsem.at[b])
        return carry

    lax.fori_loop(0, NGROUP, body, 0)
    for b in range(NBUF):
        pltpu.make_async_copy(rows_v.at[b],
                              out_hbm.at[0, i0, pl.ds(0, BLOCK)],
                              wsem.at[b]).wait()


_SC_GATHER_CACHE = []


def _sc_gather(cidx_flat, table):
    if not _SC_GATHER_CACHE:
        _SC_GATHER_CACHE.append(functools.partial(
            pl.kernel,
            mesh=plsc.VectorSubcoreMesh(core_axis_name="c",
                                        subcore_axis_name="s"),
            out_type=jax.ShapeDtypeStruct((1, L, L, D_PAIR), jnp.float32),
            scratch_types=[
                pltpu.VMEM((PER_W,), jnp.int32),
                pltpu.VMEM((NBUF, BLOCK, D_PAIR), jnp.float32),
                pltpu.VMEM_SHARED((NTAB, D_PAIR), jnp.float32),
                pltpu.SemaphoreType.DMA((NBUF,)),
                pltpu.SemaphoreType.DMA((NBUF,)),
            ],
            compiler_params=pltpu.CompilerParams(use_tc_tiling_on_sc=False),
        )(_sc_gather_body))
    return _SC_GATHER_CACHE[0](cidx_flat, table)


def kernel(seq, idx, bond_feats, same_chain, emb_res_w, emb_atom_w,
           emb_chain_w):
    seq = seq.astype(jnp.int32)
    idx = idx.astype(jnp.int32)
    bond_feats = bond_feats.astype(jnp.int32)
    same_chain = same_chain.astype(jnp.int32)
    seq_r = seq.reshape(1, L)
    seq_c = seq.reshape(L, 1)
    idx_r = idx.reshape(1, L)
    idx_c = idx.reshape(L, 1)
    cidx, table = _index_and_table(seq_r, seq_c, idx_r, idx_c, bond_feats,
                                   same_chain, emb_res_w, emb_atom_w,
                                   emb_chain_w)
    return _sc_gather(cidx.reshape(P), table)
